# baseline (speedup 1.0000x reference)
import functools

import jax
import jax.numpy as jnp
from jax import lax
from jax.experimental import pallas as pl
from jax.experimental.pallas import tpu as pltpu
from jax.experimental.pallas import tpu_sc as plsc

N_LAYERS = 4
K = 1024
DIM = 64
HALF = 256
NSUB = 8
TILE = NSUB * HALF
NCHUNK = 1

_NC = 2
_NS = 16
_NW = _NC * _NS


def _tc_layer(resid, cb, cb_norm):
    x_norm = jnp.sum(resid * resid, axis=1, keepdims=True)
    mm2 = jax.lax.dot_general(
        -2.0 * resid, cb, (((1,), (1,)), ((), ())),
        preferred_element_type=jnp.float32,
    )
    d = (x_norm + cb_norm) + mm2
    d_min = jnp.min(d, axis=1, keepdims=True)
    iota = jax.lax.broadcasted_iota(jnp.int32, d.shape, 1)
    return jnp.min(jnp.where(d == d_min, iota, K), axis=1, keepdims=True)


def _tc_body_first(x_ref, cb_ref, cbn_ref, out_ref):
    cb, cbn = cb_ref[...], cbn_ref[...]
    for s in range(NSUB):
        sl = pl.ds(s * HALF, HALF)
        out_ref[sl, :] = _tc_layer(x_ref[sl, :], cb, cbn)


def _tc_body(x_ref, delta_ref, cb_ref, cbn_ref, out_ref, resid_ref):
    cb, cbn = cb_ref[...], cbn_ref[...]
    for s in range(NSUB):
        sl = pl.ds(s * HALF, HALF)
        resid = x_ref[sl, :] - delta_ref[sl, 0:DIM]
        resid_ref[sl, :] = resid
        out_ref[sl, :] = _tc_layer(resid, cb, cbn)


def _tc_codes(resid, delta, cb, cb_norm, want_resid):
    n = resid.shape[0]
    full = lambda s: pl.BlockSpec(s, lambda i: (0,) * len(s))
    row = lambda w: pl.BlockSpec((TILE, w), lambda i: (i, 0))
    if delta is None:
        return pl.pallas_call(
            _tc_body_first,
            grid=(n // TILE,),
            in_specs=[row(DIM), full((K, DIM)), full((1, K))],
            out_specs=row(1),
            out_shape=jax.ShapeDtypeStruct((n, 1), jnp.int32),
        )(resid, cb, cb_norm)
    codes, new_resid = pl.pallas_call(
        _tc_body,
        grid=(n // TILE,),
        in_specs=[row(DIM), row(2 * DIM), full((K, DIM)), full((1, K))],
        out_specs=(row(1), row(DIM)),
        out_shape=(jax.ShapeDtypeStruct((n, 1), jnp.int32),
                   jax.ShapeDtypeStruct((n, DIM), jnp.float32)),
    )(resid, delta, cb, cb_norm)
    return (codes, new_resid) if want_resid else (codes, None)


def _sc_gather(cb_pad, codes):
    n = codes.shape[0]
    b_per_w = n // _NW
    batch = min(b_per_w, 512)
    nbatch = b_per_w // batch
    mesh = plsc.VectorSubcoreMesh(core_axis_name="c", subcore_axis_name="s")

    @functools.partial(
        pl.kernel, mesh=mesh,
        out_type=jax.ShapeDtypeStruct((n, 2 * DIM), jnp.float32),
        scratch_types=[
            pltpu.VMEM((batch,), jnp.int32),
            pltpu.VMEM((batch, 2 * DIM), jnp.float32),
            pltpu.SemaphoreType.DMA,
        ],
    )
    def body(cb_hbm, idx_hbm, out_hbm, idx_v, rows_v, sem):
        wid = lax.axis_index("s") * _NC + lax.axis_index("c")
        for t in range(nbatch):
            base = wid * b_per_w + t * batch
            pltpu.sync_copy(idx_hbm.at[pl.ds(base, batch)], idx_v)
            pltpu.async_copy(cb_hbm.at[idx_v], rows_v, sem).wait()
            pltpu.sync_copy(rows_v, out_hbm.at[pl.ds(base, batch)])

    return body(cb_pad, codes)


@jax.jit
def kernel(x, centroids):
    n = x.shape[0]
    cb_norm = jnp.sum(centroids * centroids, axis=2)
    cb_padded = jnp.pad(centroids, ((0, 0), (0, 0), (0, DIM)))
    ch = n // NCHUNK
    resids = [x[c * ch:(c + 1) * ch] for c in range(NCHUNK)]
    deltas = [None] * NCHUNK
    codes = [[] for _ in range(NCHUNK)]
    for l in range(N_LAYERS):
        cb = centroids[l]
        cbn = cb_norm[l][None, :]
        for c in range(NCHUNK):
            if l == 0:
                code = _tc_codes(resids[c], None, cb, cbn, False)
            else:
                code, r = _tc_codes(resids[c], deltas[c], cb, cbn,
                                    l + 1 < N_LAYERS)
                if r is not None:
                    resids[c] = r
            codes[c].append(code)
            if l + 1 < N_LAYERS:
                deltas[c] = _sc_gather(cb_padded[l], code[:, 0])
    return jnp.concatenate(
        [jnp.concatenate(codes[c], axis=1) for c in range(NCHUNK)], axis=0)


# speedup vs baseline: 1.3438x; 1.3438x over previous
import functools

import jax
import jax.numpy as jnp
from jax.experimental import pallas as pl

N_LAYERS = 4
K = 1024
DIM = 64
HALF = 256
NSUB = 8
TILE = NSUB * HALF


def _layer(resid, cb, cb_norm, cbsplit, iota):
    mm2 = jax.lax.dot_general(
        -2.0 * resid, cb, (((1,), (1,)), ((), ())),
        preferred_element_type=jnp.float32,
    )
    d = cb_norm + mm2
    d_min = jnp.min(d, axis=1, keepdims=True)
    code = jnp.min(jnp.where(d == d_min, iota, float(K)), axis=1,
                   keepdims=True)
    onehot = (iota == code).astype(jnp.bfloat16)
    dn = (((1,), (0,)), ((), ()))
    d3 = jax.lax.dot_general(onehot, cbsplit, dn,
                             preferred_element_type=jnp.float32)
    delta = ((d3[:, DIM:2 * DIM] + d3[:, 2 * DIM:3 * DIM]) + d3[:, :DIM]) \
        + d3[:, 3 * DIM:]
    return resid - delta, code.astype(jnp.int32)


def _body(x_ref, cb_ref, cbn_ref, cbsplit_ref, iota_ref, out_ref):
    resids = [x_ref[s * HALF:(s + 1) * HALF] for s in range(NSUB)]
    codes = [[] for _ in range(NSUB)]
    iota = iota_ref[...]
    for l in range(N_LAYERS):
        cb, cbn, cbsplit = cb_ref[l], cbn_ref[l][None, :], cbsplit_ref[l]
        for s in range(NSUB):
            resids[s], code = _layer(resids[s], cb, cbn, cbsplit, iota)
            codes[s].append(code)
    for s in range(NSUB):
        out_ref[s * HALF:(s + 1) * HALF, :] = jnp.concatenate(codes[s], axis=1)


@jax.jit
def kernel(x, centroids):
    n = x.shape[0]
    cb_norm = jnp.sum(centroids * centroids, axis=2)
    hi = centroids.astype(jnp.bfloat16)
    r1 = centroids - hi.astype(jnp.float32)
    mid = r1.astype(jnp.bfloat16)
    r2 = r1 - mid.astype(jnp.float32)
    lo = r2.astype(jnp.bfloat16)
    lo2 = (r2 - lo.astype(jnp.float32)).astype(jnp.bfloat16)
    cbsplit = jnp.concatenate([hi, mid, lo, lo2], axis=2)
    full = lambda s: pl.BlockSpec(s, lambda i: (0,) * len(s))
    return pl.pallas_call(
        _body,
        grid=(n // TILE,),
        in_specs=[
            pl.BlockSpec((TILE, DIM), lambda i: (i, 0)),
            full((N_LAYERS, K, DIM)),
            full((N_LAYERS, K)),
            full((N_LAYERS, K, 4 * DIM)),
            full((1, K)),
        ],
        out_specs=pl.BlockSpec((TILE, N_LAYERS), lambda i: (i, 0)),
        out_shape=jax.ShapeDtypeStruct((n, N_LAYERS), jnp.int32),
    )(x, centroids, cb_norm, cbsplit,
      jnp.arange(K, dtype=jnp.float32).reshape(1, K))
